# baseline (speedup 1.0000x reference)
import functools

import jax
import jax.numpy as jnp
from jax import lax
from jax.experimental import pallas as pl
from jax.experimental.pallas import tpu as pltpu
from jax.experimental.pallas import tpu_sc as plsc

VOCAB = 1000000
D = 64
SEQ = 200
BATCH = 4096

_info = plsc.get_sparse_core_info()
NC = _info.num_cores
NS = _info.num_subcores
LANES = _info.num_lanes
NW = NC * NS

ROWS = BATCH * SEQ
ROWS_PER_W = ROWS // NW
CHUNK = 128
CHUNKS = ROWS_PER_W // CHUNK
POS_FLOATS = SEQ * D


def _body(x_hbm, emb_hbm, pos2_hbm, out_hbm, idx_v, pos_v, buf_v, sem):
    wid = lax.axis_index("s") * NC + lax.axis_index("c")
    pltpu.sync_copy(x_hbm.at[wid], idx_v)
    pltpu.sync_copy(pos2_hbm, pos_v)

    def chunk_body(j, carry):
        pltpu.async_copy(emb_hbm.at[idx_v.at[j]], buf_v, sem).wait()
        start = (j * (CHUNK * D)) % POS_FLOATS

        def row_body(r, c2):
            off = start + r * D
            for c in range(D // LANES):
                sl = pl.ds(c * LANES, LANES)
                buf_v[r, sl] = buf_v[r, sl] + pos_v[pl.ds(off + c * LANES, LANES)]
            return c2

        lax.fori_loop(0, CHUNK, row_body, 0)
        base = wid * ROWS_PER_W + j * CHUNK
        pltpu.sync_copy(buf_v, out_hbm.at[pl.ds(base, CHUNK)])
        return carry

    lax.fori_loop(0, CHUNKS, chunk_body, 0)


@functools.partial(jax.jit, static_argnums=())
def _sc_embed(x3, embedding, pos2):
    mesh = plsc.VectorSubcoreMesh(core_axis_name="c", subcore_axis_name="s")
    f = pl.kernel(
        _body,
        mesh=mesh,
        compiler_params=pltpu.CompilerParams(use_tc_tiling_on_sc=False),
        out_type=jax.ShapeDtypeStruct((ROWS, D), jnp.float32),
        scratch_types=[
            pltpu.VMEM((CHUNKS, CHUNK), jnp.int32),
            pltpu.VMEM((2 * POS_FLOATS,), jnp.float32),
            pltpu.VMEM((CHUNK, D), jnp.float32),
            pltpu.SemaphoreType.DMA,
        ],
    )
    return f(x3, embedding, pos2)


def kernel(x, embedding, pos_embedding):
    x3 = x.astype(jnp.int32).reshape(NW, CHUNKS, CHUNK)
    pos_flat = pos_embedding.reshape(-1)
    pos2 = jnp.concatenate([pos_flat, pos_flat])
    out = _sc_embed(x3, embedding, pos2)
    return out.reshape(BATCH, SEQ, D)


# speedup vs baseline: 1.5391x; 1.5391x over previous
import functools

import jax
import jax.numpy as jnp
from jax import lax
from jax.experimental import pallas as pl
from jax.experimental.pallas import tpu as pltpu
from jax.experimental.pallas import tpu_sc as plsc

VOCAB = 1000000
D = 64
SEQ = 200
BATCH = 4096

_info = plsc.get_sparse_core_info()
NC = _info.num_cores
NS = _info.num_subcores
LANES = _info.num_lanes
NW = NC * NS

ROWS = BATCH * SEQ
ROWS_PER_W = ROWS // NW
CHUNK = 128
CHUNKS = ROWS_PER_W // CHUNK
POS_FLOATS = SEQ * D
NBUF = 4
LOOKAHEAD = NBUF - 1


def _body(x_hbm, emb_hbm, pos2_hbm, out_hbm, idx_v, pos_v, bufs, gsem, ssem):
    wid = lax.axis_index("s") * NC + lax.axis_index("c")
    pltpu.sync_copy(x_hbm.at[wid], idx_v)
    pltpu.sync_copy(pos2_hbm, pos_v)
    out_base = wid * ROWS_PER_W

    def start_gather(j, b):
        pltpu.async_copy(emb_hbm.at[idx_v.at[j]], bufs.at[b], gsem)

    for t in range(LOOKAHEAD):
        start_gather(t, t)

    def chunk_body(j, carry):
        b = lax.rem(j, NBUF)
        pltpu.make_async_copy(emb_hbm.at[idx_v.at[j]], bufs.at[b], gsem).wait()
        start = lax.rem(j * (CHUNK * D), POS_FLOATS)

        @plsc.parallel_loop(0, CHUNK, unroll=4)
        def _(r):
            off = start + r * D
            for c in range(D // LANES):
                plsc.addupdate(
                    bufs.at[b, r, pl.ds(c * LANES, LANES)],
                    pos_v[pl.ds(off + c * LANES, LANES)],
                )

        pltpu.async_copy(
            bufs.at[b], out_hbm.at[pl.ds(out_base + j * CHUNK, CHUNK)], ssem
        )

        @pl.when(jnp.logical_and(j + LOOKAHEAD < CHUNKS, j >= 1))
        def _():
            pltpu.make_async_copy(
                bufs.at[0], out_hbm.at[pl.ds(out_base, CHUNK)], ssem
            ).wait()

        @pl.when(j + LOOKAHEAD < CHUNKS)
        def _():
            start_gather(j + LOOKAHEAD, lax.rem(j + LOOKAHEAD, NBUF))

        return carry

    lax.fori_loop(0, CHUNKS, chunk_body, 0)

    for _ in range(NBUF):
        pltpu.make_async_copy(
            bufs.at[0], out_hbm.at[pl.ds(out_base, CHUNK)], ssem
        ).wait()


@jax.jit
def _sc_embed(x3, embedding, pos2):
    mesh = plsc.VectorSubcoreMesh(core_axis_name="c", subcore_axis_name="s")
    f = pl.kernel(
        _body,
        mesh=mesh,
        compiler_params=pltpu.CompilerParams(use_tc_tiling_on_sc=False),
        out_type=jax.ShapeDtypeStruct((ROWS, D), jnp.float32),
        scratch_types=[
            pltpu.VMEM((CHUNKS, CHUNK), jnp.int32),
            pltpu.VMEM((2 * POS_FLOATS,), jnp.float32),
            pltpu.VMEM((NBUF, CHUNK, D), jnp.float32),
            pltpu.SemaphoreType.DMA,
            pltpu.SemaphoreType.DMA,
        ],
    )
    return f(x3, embedding, pos2)


def kernel(x, embedding, pos_embedding):
    x3 = x.astype(jnp.int32).reshape(NW, CHUNKS, CHUNK)
    pos_flat = pos_embedding.reshape(-1)
    pos2 = jnp.concatenate([pos_flat, pos_flat])
    out = _sc_embed(x3, embedding, pos2)
    return out.reshape(BATCH, SEQ, D)


# speedup vs baseline: 1.7064x; 1.1087x over previous
import jax
import jax.numpy as jnp
from jax import lax
from jax.experimental import pallas as pl
from jax.experimental.pallas import tpu as pltpu
from jax.experimental.pallas import tpu_sc as plsc

VOCAB = 1000000
D = 64
SEQ = 200
BATCH = 4096

_info = plsc.get_sparse_core_info()
NC = _info.num_cores
NS = _info.num_subcores
LANES = _info.num_lanes
NW = NC * NS

ABLK = 128
NFULL = VOCAB // ABLK
TAIL = VOCAB - NFULL * ABLK
A_PER_W = -(-NFULL // NW)

CHUNK = 128
CBLK = BATCH // CHUNK
UNITS = SEQ * CBLK
U_PER_W = UNITS // NW
OUT_ROWS = SEQ * D * BATCH // 128


def _body_a(embt_hbm, tail_hbm, flat_hbm, blk, tblk, tailv, isem, osem):
    wid = lax.axis_index("s") * NC + lax.axis_index("c")

    @pl.when(wid == NW - 1)
    def _():
        pltpu.sync_copy(tail_hbm, tailv)
        pltpu.sync_copy(tailv, flat_hbm.at[pl.ds(NFULL * ABLK * D, TAIL * D)])

    row_idx = [lax.iota(jnp.int32, 16) + (16 * g) for g in range(D // LANES)]

    def blk_id(t):
        return wid + t * NW

    def start_in(t, b):
        pltpu.async_copy(
            embt_hbm.at[:, pl.ds(blk_id(t) * ABLK, ABLK)],
            blk.at[b, :, pl.ds(0, ABLK)],
            isem,
        )

    @pl.when(blk_id(0) < NFULL)
    def _():
        start_in(0, 0)

    def step(t, carry):
        b = lax.rem(t, 2)

        @pl.when(blk_id(t) < NFULL)
        def _():
            pltpu.make_async_copy(
                embt_hbm.at[:, pl.ds(blk_id(t) * ABLK, ABLK)],
                blk.at[b, :, pl.ds(0, ABLK)],
                isem,
            ).wait()

            @pl.when(blk_id(t + 1) < NFULL)
            def _():
                start_in(t + 1, 1 - b)

            @pl.when(t >= 2)
            def _():
                pltpu.make_async_copy(
                    tblk.at[0], flat_hbm.at[pl.ds(0, ABLK * D)], osem
                ).wait()

            @plsc.parallel_loop(0, ABLK, unroll=2)
            def _(j):
                cols = jnp.full((16,), j, jnp.int32)
                for g in range(D // LANES):
                    vals = plsc.load_gather(blk.at[b], [row_idx[g], cols])
                    tblk[b, pl.ds(j * D + 16 * g, 16)] = vals

            pltpu.async_copy(
                tblk.at[b], flat_hbm.at[pl.ds(blk_id(t) * ABLK * D, ABLK * D)], osem
            )

        return carry

    lax.fori_loop(0, A_PER_W, step, 0)

    nblk = jnp.minimum(
        jnp.maximum(NFULL - wid + NW - 1, 0) // NW, A_PER_W
    ).astype(jnp.int32)

    @pl.when(nblk >= 2)
    def _():
        pltpu.make_async_copy(
            tblk.at[0], flat_hbm.at[pl.ds(0, ABLK * D)], osem
        ).wait()
        pltpu.make_async_copy(
            tblk.at[0], flat_hbm.at[pl.ds(0, ABLK * D)], osem
        ).wait()

    @pl.when(nblk == 1)
    def _():
        pltpu.make_async_copy(
            tblk.at[0], flat_hbm.at[pl.ds(0, ABLK * D)], osem
        ).wait()


def _body_b(xt_hbm, table_hbm, pos_hbm, out_hbm,
            pos_v, idxb, rows, tile, isem, gsem, ssem):
    wid = lax.axis_index("s") * NC + lax.axis_index("c")
    u0 = wid * U_PER_W
    pltpu.sync_copy(pos_hbm, pos_v)
    f_idx = [lax.iota(jnp.int32, 16) + (16 * g) for g in range(D // LANES)]

    def unit_lc(u):
        return lax.div(u, CBLK), lax.rem(u, CBLK)

    def start_idx(u, b):
        l, c = unit_lc(u0 + u)
        pltpu.async_copy(
            xt_hbm.at[l, pl.ds(c * CHUNK, CHUNK)], idxb.at[b], isem
        )

    def gather_pair(u, b):
        return (table_hbm.at[idxb.at[b]], rows.at[b])

    def start_gather(u, b):
        src, dst = gather_pair(u, b)
        pltpu.async_copy(src, dst, gsem)

    l0, c0 = unit_lc(u0)
    pltpu.sync_copy(xt_hbm.at[l0, pl.ds(c0 * CHUNK, CHUNK)], idxb.at[0])
    start_gather(0, 0)
    start_idx(1, 1)

    def step(u, carry):
        b = lax.rem(u, 2)
        l, _c = unit_lc(u0 + u)
        src, dst = gather_pair(u, b)
        pltpu.make_async_copy(src, dst, gsem).wait()

        @pl.when(u + 1 < U_PER_W)
        def _():
            pltpu.make_async_copy(
                xt_hbm.at[0, pl.ds(0, CHUNK)], idxb.at[1 - b], isem
            ).wait()
            start_gather(u + 1, 1 - b)

        @pl.when(u + 2 < U_PER_W)
        def _():
            start_idx(u + 2, b)

        @pl.when(u >= 2)
        def _():
            pltpu.make_async_copy(
                tile.at[0, :, pl.ds(0, CHUNK)], out_hbm.at[pl.ds(0, D)], ssem
            ).wait()

        pv = [pos_v[l, pl.ds(16 * g, 16)] for g in range(D // LANES)]

        @plsc.parallel_loop(0, CHUNK, unroll=2)
        def _(j):
            cols = jnp.full((16,), j, jnp.int32)
            for g in range(D // LANES):
                vals = rows[b, j, pl.ds(16 * g, 16)] + pv[g]
                plsc.store_scatter(tile.at[b], [f_idx[g], cols], vals)

        base = l * (8 * CBLK * 8) + _c * 8
        for r in range(8):
            pltpu.async_copy(
                tile.at[b, pl.ds(r * 8, 8), pl.ds(0, CHUNK)],
                out_hbm.at[pl.ds(base + r * (CBLK * 8), 8)],
                ssem,
            )
        return carry

    lax.fori_loop(0, U_PER_W, step, 0)

    for _ in range(2):
        pltpu.make_async_copy(
            tile.at[0, :, pl.ds(0, CHUNK)], out_hbm.at[pl.ds(0, D)], ssem
        ).wait()


@jax.jit
def _sc_embed(embt, tail_flat, xt, pos):
    mesh = plsc.VectorSubcoreMesh(core_axis_name="c", subcore_axis_name="s")
    fa = pl.kernel(
        _body_a,
        mesh=mesh,
        compiler_params=pltpu.CompilerParams(use_tc_tiling_on_sc=True, needs_layout_passes=False),
        out_type=jax.ShapeDtypeStruct((VOCAB * D,), jnp.float32),
        scratch_types=[
            pltpu.VMEM((2, D, ABLK + 1), jnp.float32),
            pltpu.VMEM((2, ABLK * D), jnp.float32),
            pltpu.VMEM((TAIL * D,), jnp.float32),
            pltpu.SemaphoreType.DMA,
            pltpu.SemaphoreType.DMA,
        ],
    )
    flat_table = fa(embt, tail_flat)
    table = flat_table.reshape(VOCAB, D)

    fb = pl.kernel(
        _body_b,
        mesh=mesh,
        compiler_params=pltpu.CompilerParams(use_tc_tiling_on_sc=False, needs_layout_passes=False),
        out_type=jax.ShapeDtypeStruct((OUT_ROWS, 128), jnp.float32),
        scratch_types=[
            pltpu.VMEM((SEQ, D), jnp.float32),
            pltpu.VMEM((2, CHUNK), jnp.int32),
            pltpu.VMEM((2, CHUNK, D), jnp.float32),
            pltpu.VMEM((2, D, CHUNK + 1), jnp.float32),
            pltpu.SemaphoreType.DMA,
            pltpu.SemaphoreType.DMA,
            pltpu.SemaphoreType.DMA,
        ],
    )
    out2 = fb(xt, table, pos)
    out5 = out2.reshape(SEQ, 8, CBLK, 8, 128)
    return out5.transpose(2, 4, 0, 1, 3).reshape(BATCH, SEQ, D)


def kernel(x, embedding, pos_embedding):
    embt = embedding.T
    tail_flat = embedding[NFULL * ABLK :].reshape(-1)
    xt = x.T.astype(jnp.int32)
    return _sc_embed(embt, tail_flat, xt, pos_embedding)


# speedup vs baseline: 2.9005x; 1.6998x over previous
import jax
import jax.numpy as jnp
from jax import lax
from jax.experimental import pallas as pl
from jax.experimental.pallas import tpu as pltpu
from jax.experimental.pallas import tpu_sc as plsc

VOCAB = 1000000
D = 64
SEQ = 200
BATCH = 4096

_info = plsc.get_sparse_core_info()
NC = _info.num_cores
NS = _info.num_subcores
LANES = _info.num_lanes
NW = NC * NS

ABLK = 128
NFULL = VOCAB // ABLK
TAIL = VOCAB - NFULL * ABLK
A_PER_W = -(-NFULL // NW)

CHUNK = 128
CBLK = BATCH // CHUNK
UNITS = SEQ * CBLK
U_PER_W = UNITS // NW
OUT_ROWS = SEQ * D * BATCH // 128


def _body_a(embt_hbm, tail_hbm, flat_hbm, blk, tblk, tailv, isem, osem):
    wid = lax.axis_index("s") * NC + lax.axis_index("c")

    @pl.when(wid == NW - 1)
    def _():
        pltpu.sync_copy(tail_hbm, tailv)
        pltpu.sync_copy(tailv, flat_hbm.at[pl.ds(NFULL * ABLK * D, TAIL * D)])

    iota = lax.iota(jnp.int32, 16)
    rows_f = [iota + (16 * g) for g in range(D // LANES)]
    rot = [lax.rem(iota + d, 16) for d in range(LANES)]

    def blk_id(t):
        return wid + t * NW

    def start_in(t, b):
        pltpu.async_copy(
            embt_hbm.at[:, pl.ds(blk_id(t) * ABLK, ABLK)],
            blk.at[b, :, pl.ds(0, ABLK)],
            isem,
        )

    @pl.when(blk_id(0) < NFULL)
    def _():
        start_in(0, 0)

    def step(t, carry):
        b = lax.rem(t, 2)

        @pl.when(blk_id(t) < NFULL)
        def _():
            pltpu.make_async_copy(
                embt_hbm.at[:, pl.ds(blk_id(t) * ABLK, ABLK)],
                blk.at[b, :, pl.ds(0, ABLK)],
                isem,
            ).wait()

            @pl.when(blk_id(t + 1) < NFULL)
            def _():
                start_in(t + 1, 1 - b)

            @pl.when(t >= 2)
            def _():
                pltpu.make_async_copy(
                    tblk.at[0], flat_hbm.at[pl.ds(0, ABLK * D)], osem
                ).wait()

            bvec = jnp.full((16,), b, jnp.int32)

            @plsc.parallel_loop(0, ABLK // LANES)
            def _(j16):
                for d in range(LANES):
                    cols = j16 * 16 + rot[d]
                    colsx = lax.shift_left(cols, 6)
                    for g in range(D // LANES):
                        vals = plsc.load_gather(blk.at[b], [rows_f[g], cols])
                        plsc.store_scatter(
                            tblk, [bvec, colsx + rows_f[g]], vals
                        )

            pltpu.async_copy(
                tblk.at[b], flat_hbm.at[pl.ds(blk_id(t) * ABLK * D, ABLK * D)], osem
            )

        return carry

    lax.fori_loop(0, A_PER_W, step, 0)

    nblk = jnp.minimum(
        jnp.maximum(NFULL - wid + NW - 1, 0) // NW, A_PER_W
    ).astype(jnp.int32)

    @pl.when(nblk >= 2)
    def _():
        pltpu.make_async_copy(
            tblk.at[0], flat_hbm.at[pl.ds(0, ABLK * D)], osem
        ).wait()
        pltpu.make_async_copy(
            tblk.at[0], flat_hbm.at[pl.ds(0, ABLK * D)], osem
        ).wait()

    @pl.when(nblk == 1)
    def _():
        pltpu.make_async_copy(
            tblk.at[0], flat_hbm.at[pl.ds(0, ABLK * D)], osem
        ).wait()


def _body_b(xt_hbm, table_hbm, pos_hbm, out_hbm,
            pos_v, idxb, rows, tile, isem, gsem, ssem):
    wid = lax.axis_index("s") * NC + lax.axis_index("c")
    u0 = wid * U_PER_W
    pltpu.sync_copy(pos_hbm, pos_v)
    f_idx = [lax.iota(jnp.int32, 16) + (16 * g) for g in range(D // LANES)]

    def unit_lc(u):
        return lax.div(u, CBLK), lax.rem(u, CBLK)

    def start_idx(u, b):
        l, c = unit_lc(u0 + u)
        pltpu.async_copy(
            xt_hbm.at[l, pl.ds(c * CHUNK, CHUNK)], idxb.at[b], isem
        )

    def gather_pair(u, b):
        return (table_hbm.at[idxb.at[b]], rows.at[b])

    def start_gather(u, b):
        src, dst = gather_pair(u, b)
        pltpu.async_copy(src, dst, gsem)

    l0, c0 = unit_lc(u0)
    pltpu.sync_copy(xt_hbm.at[l0, pl.ds(c0 * CHUNK, CHUNK)], idxb.at[0])
    start_gather(0, 0)
    start_idx(1, 1)

    def step(u, carry):
        b = lax.rem(u, 2)
        l, _c = unit_lc(u0 + u)
        src, dst = gather_pair(u, b)
        pltpu.make_async_copy(src, dst, gsem).wait()

        @pl.when(u + 1 < U_PER_W)
        def _():
            pltpu.make_async_copy(
                xt_hbm.at[0, pl.ds(0, CHUNK)], idxb.at[1 - b], isem
            ).wait()
            start_gather(u + 1, 1 - b)

        @pl.when(u + 2 < U_PER_W)
        def _():
            start_idx(u + 2, b)

        @pl.when(u >= 2)
        def _():
            pltpu.make_async_copy(
                tile.at[0, :, pl.ds(0, CHUNK)], out_hbm.at[pl.ds(0, D)], ssem
            ).wait()

        pv = [pos_v[l, pl.ds(16 * g, 16)] for g in range(D // LANES)]

        @plsc.parallel_loop(0, CHUNK, unroll=2)
        def _(j):
            cols = jnp.full((16,), j, jnp.int32)
            for g in range(D // LANES):
                vals = rows[b, j, pl.ds(16 * g, 16)] + pv[g]
                plsc.store_scatter(tile.at[b], [f_idx[g], cols], vals)

        base = l * (8 * CBLK * 8) + _c * 8
        for r in range(8):
            pltpu.async_copy(
                tile.at[b, pl.ds(r * 8, 8), pl.ds(0, CHUNK)],
                out_hbm.at[pl.ds(base + r * (CBLK * 8), 8)],
                ssem,
            )
        return carry

    lax.fori_loop(0, U_PER_W, step, 0)

    for _ in range(2):
        pltpu.make_async_copy(
            tile.at[0, :, pl.ds(0, CHUNK)], out_hbm.at[pl.ds(0, D)], ssem
        ).wait()


@jax.jit
def _sc_embed(embt, tail_flat, xt, pos):
    mesh = plsc.VectorSubcoreMesh(core_axis_name="c", subcore_axis_name="s")
    fa = pl.kernel(
        _body_a,
        mesh=mesh,
        compiler_params=pltpu.CompilerParams(use_tc_tiling_on_sc=True, needs_layout_passes=False),
        out_type=jax.ShapeDtypeStruct((VOCAB * D,), jnp.float32),
        scratch_types=[
            pltpu.VMEM((2, D, ABLK), jnp.float32),
            pltpu.VMEM((2, ABLK * D), jnp.float32),
            pltpu.VMEM((TAIL * D,), jnp.float32),
            pltpu.SemaphoreType.DMA,
            pltpu.SemaphoreType.DMA,
        ],
    )
    flat_table = fa(embt, tail_flat)
    table = flat_table.reshape(VOCAB, D)

    fb = pl.kernel(
        _body_b,
        mesh=mesh,
        compiler_params=pltpu.CompilerParams(use_tc_tiling_on_sc=False, needs_layout_passes=False),
        out_type=jax.ShapeDtypeStruct((OUT_ROWS, 128), jnp.float32),
        scratch_types=[
            pltpu.VMEM((SEQ, D), jnp.float32),
            pltpu.VMEM((2, CHUNK), jnp.int32),
            pltpu.VMEM((2, CHUNK, D), jnp.float32),
            pltpu.VMEM((2, D, CHUNK + 1), jnp.float32),
            pltpu.SemaphoreType.DMA,
            pltpu.SemaphoreType.DMA,
            pltpu.SemaphoreType.DMA,
        ],
    )
    out2 = fb(xt, table, pos)
    out5 = out2.reshape(SEQ, 8, CBLK, 8, 128)
    return out5.transpose(2, 4, 0, 1, 3).reshape(BATCH, SEQ, D)


def kernel(x, embedding, pos_embedding):
    embt = embedding.T
    tail_flat = embedding[NFULL * ABLK :].reshape(-1)
    xt = x.T.astype(jnp.int32)
    return _sc_embed(embt, tail_flat, xt, pos_embedding)


# speedup vs baseline: 3.6518x; 1.2590x over previous
import jax
import jax.numpy as jnp
from jax import lax
from jax.experimental import pallas as pl
from jax.experimental.pallas import tpu as pltpu
from jax.experimental.pallas import tpu_sc as plsc

VOCAB = 1000000
D = 64
SEQ = 200
BATCH = 4096

_info = plsc.get_sparse_core_info()
NC = _info.num_cores
NS = _info.num_subcores
LANES = _info.num_lanes
NW = NC * NS

ABLK = 256
NFULL = VOCAB // ABLK
TAIL = VOCAB - NFULL * ABLK
A_PER_W = -(-NFULL // NW)

CHUNK = 128
CBLK = BATCH // CHUNK
UNITS = SEQ * CBLK
U_PER_W = UNITS // NW
OUT_ROWS = SEQ * D * BATCH // 128


def _body_a(embt_hbm, tail_hbm, flat_hbm, blk, tblk, tailv, isem, osem):
    wid = lax.axis_index("s") * NC + lax.axis_index("c")

    @pl.when(wid == NW - 1)
    def _():
        pltpu.sync_copy(tail_hbm, tailv)
        pltpu.sync_copy(tailv, flat_hbm.at[pl.ds(NFULL * ABLK * D, TAIL * D)])

    iota = lax.iota(jnp.int32, 16)
    rows_f = [iota + (16 * g) for g in range(D // LANES)]
    rot = [lax.rem(iota + d, 16) for d in range(LANES)]

    def blk_id(t):
        return wid + t * NW

    def start_in(t, b):
        pltpu.async_copy(
            embt_hbm.at[:, pl.ds(blk_id(t) * ABLK, ABLK)],
            blk.at[b, :, pl.ds(0, ABLK)],
            isem,
        )

    @pl.when(blk_id(0) < NFULL)
    def _():
        start_in(0, 0)

    def step(t, carry):
        b = lax.rem(t, 2)

        @pl.when(blk_id(t) < NFULL)
        def _():
            pltpu.make_async_copy(
                embt_hbm.at[:, pl.ds(blk_id(t) * ABLK, ABLK)],
                blk.at[b, :, pl.ds(0, ABLK)],
                isem,
            ).wait()

            @pl.when(blk_id(t + 1) < NFULL)
            def _():
                start_in(t + 1, 1 - b)

            @pl.when(t >= 2)
            def _():
                pltpu.make_async_copy(
                    tblk.at[0], flat_hbm.at[pl.ds(0, ABLK * D)], osem
                ).wait()

            bvec = jnp.full((16,), b, jnp.int32)

            @plsc.parallel_loop(0, ABLK // LANES)
            def _(j16):
                for d in range(LANES):
                    cols = j16 * 16 + rot[d]
                    colsx = lax.shift_left(cols, 6)
                    for g in range(D // LANES):
                        vals = plsc.load_gather(blk.at[b], [rows_f[g], cols])
                        plsc.store_scatter(
                            tblk, [bvec, colsx + rows_f[g]], vals
                        )

            pltpu.async_copy(
                tblk.at[b], flat_hbm.at[pl.ds(blk_id(t) * ABLK * D, ABLK * D)], osem
            )

        return carry

    lax.fori_loop(0, A_PER_W, step, 0)

    nblk = jnp.minimum(
        jnp.maximum(NFULL - wid + NW - 1, 0) // NW, A_PER_W
    ).astype(jnp.int32)

    @pl.when(nblk >= 2)
    def _():
        pltpu.make_async_copy(
            tblk.at[0], flat_hbm.at[pl.ds(0, ABLK * D)], osem
        ).wait()
        pltpu.make_async_copy(
            tblk.at[0], flat_hbm.at[pl.ds(0, ABLK * D)], osem
        ).wait()

    @pl.when(nblk == 1)
    def _():
        pltpu.make_async_copy(
            tblk.at[0], flat_hbm.at[pl.ds(0, ABLK * D)], osem
        ).wait()


def _body_b(xt_hbm, table_hbm, pos_hbm, out_hbm,
            pos_v, idxb, rows, tile, isem, gsem, ssem):
    wid = lax.axis_index("s") * NC + lax.axis_index("c")
    u0 = wid * U_PER_W
    pltpu.sync_copy(pos_hbm, pos_v)
    f_idx = [lax.iota(jnp.int32, 16) + (16 * g) for g in range(D // LANES)]

    def unit_lc(u):
        return lax.div(u, CBLK), lax.rem(u, CBLK)

    def start_idx(u, b):
        l, c = unit_lc(u0 + u)
        pltpu.async_copy(
            xt_hbm.at[l, pl.ds(c * CHUNK, CHUNK)], idxb.at[b], isem
        )

    def gather_pair(u, b):
        return (table_hbm.at[idxb.at[b]], rows.at[b])

    def start_gather(u, b):
        src, dst = gather_pair(u, b)
        pltpu.async_copy(src, dst, gsem)

    l0, c0 = unit_lc(u0)
    pltpu.sync_copy(xt_hbm.at[l0, pl.ds(c0 * CHUNK, CHUNK)], idxb.at[0])
    start_gather(0, 0)
    start_idx(1, 1)

    def step(u, carry):
        b = lax.rem(u, 2)
        l, _c = unit_lc(u0 + u)
        src, dst = gather_pair(u, b)
        pltpu.make_async_copy(src, dst, gsem).wait()

        @pl.when(u + 1 < U_PER_W)
        def _():
            pltpu.make_async_copy(
                xt_hbm.at[0, pl.ds(0, CHUNK)], idxb.at[1 - b], isem
            ).wait()
            start_gather(u + 1, 1 - b)

        @pl.when(u + 2 < U_PER_W)
        def _():
            start_idx(u + 2, b)

        @pl.when(u >= 2)
        def _():
            pltpu.make_async_copy(
                tile.at[0, :, pl.ds(0, CHUNK)], out_hbm.at[pl.ds(0, D)], ssem
            ).wait()

        pv = [pos_v[l, pl.ds(16 * g, 16)] for g in range(D // LANES)]

        @plsc.parallel_loop(0, CHUNK, unroll=2)
        def _(j):
            cols = jnp.full((16,), j, jnp.int32)
            for g in range(D // LANES):
                vals = rows[b, j, pl.ds(16 * g, 16)] + pv[g]
                plsc.store_scatter(tile.at[b], [f_idx[g], cols], vals)

        base = l * (8 * CBLK * 8) + _c * 8
        for r in range(8):
            pltpu.async_copy(
                tile.at[b, pl.ds(r * 8, 8), pl.ds(0, CHUNK)],
                out_hbm.at[pl.ds(base + r * (CBLK * 8), 8)],
                ssem,
            )
        return carry

    lax.fori_loop(0, U_PER_W, step, 0)

    for _ in range(2):
        pltpu.make_async_copy(
            tile.at[0, :, pl.ds(0, CHUNK)], out_hbm.at[pl.ds(0, D)], ssem
        ).wait()


@jax.jit
def _sc_embed(embt, tail_flat, xt, pos):
    mesh = plsc.VectorSubcoreMesh(core_axis_name="c", subcore_axis_name="s")
    fa = pl.kernel(
        _body_a,
        mesh=mesh,
        compiler_params=pltpu.CompilerParams(use_tc_tiling_on_sc=True, needs_layout_passes=False),
        out_type=jax.ShapeDtypeStruct((VOCAB * D,), jnp.float32),
        scratch_types=[
            pltpu.VMEM((2, D, ABLK), jnp.float32),
            pltpu.VMEM((2, ABLK * D), jnp.float32),
            pltpu.VMEM((TAIL * D,), jnp.float32),
            pltpu.SemaphoreType.DMA,
            pltpu.SemaphoreType.DMA,
        ],
    )
    flat_table = fa(embt, tail_flat)
    table = flat_table.reshape(VOCAB, D)

    fb = pl.kernel(
        _body_b,
        mesh=mesh,
        compiler_params=pltpu.CompilerParams(use_tc_tiling_on_sc=False, needs_layout_passes=False),
        out_type=jax.ShapeDtypeStruct((OUT_ROWS, 128), jnp.float32),
        scratch_types=[
            pltpu.VMEM((SEQ, D), jnp.float32),
            pltpu.VMEM((2, CHUNK), jnp.int32),
            pltpu.VMEM((2, CHUNK, D), jnp.float32),
            pltpu.VMEM((2, D, CHUNK + 1), jnp.float32),
            pltpu.SemaphoreType.DMA,
            pltpu.SemaphoreType.DMA,
            pltpu.SemaphoreType.DMA,
        ],
    )
    out2 = fb(xt, table, pos)
    out5 = out2.reshape(SEQ, 8, CBLK, 8, 128)
    return out5.transpose(2, 4, 0, 1, 3).reshape(BATCH, SEQ, D)


def kernel(x, embedding, pos_embedding):
    embt = embedding.T
    tail_flat = embedding[NFULL * ABLK :].reshape(-1)
    xt = x.T.astype(jnp.int32)
    return _sc_embed(embt, tail_flat, xt, pos_embedding)
